# baseline (speedup 1.0000x reference)
import functools

import jax
import jax.numpy as jnp
import numpy as np
from jax import lax
from jax.experimental import pallas as pl
from jax.experimental.pallas import tpu as pltpu
from jax.experimental.pallas import tpu_sc as plsc

NUM_FIELDS = 26
VOCAB = 100000
EMB_DIM = 128
BATCH = 4096

NC, NS, LANES = 2, 16, 16
NW = NC * NS
TOTAL_ROWS = BATCH * NUM_FIELDS
ROWS_PER_W = TOTAL_ROWS // NW
BLK = 128
BLKS_PER_W = ROWS_PER_W // BLK
NBUF = 7


def _body(tab_hbm, idx_hbm, offs_hbm, out_hbm, idx_v, offs_v, rows0, rows1,
          rows2, rows3, rows4, rows5, rows6, gsem, osem):
    wid = lax.axis_index("s") * NC + lax.axis_index("c")
    base = wid * ROWS_PER_W

    c1 = pltpu.async_copy(idx_hbm.at[wid], idx_v, gsem)
    c2 = pltpu.async_copy(offs_hbm.at[wid], offs_v, gsem)
    c1.wait()
    c2.wait()

    def add_block(g):
        for c in range(BLK // LANES):
            sl = pl.ds(g * BLK + c * LANES, LANES)
            idx_v[sl] = idx_v[sl] + offs_v[sl]

    bufs = (rows0, rows1, rows2, rows3, rows4, rows5, rows6)

    def gather(g, slot):
        return pltpu.async_copy(
            tab_hbm.at[idx_v.at[pl.ds(g * BLK, BLK)]], bufs[slot], gsem)

    gathers = {}
    outs = {}
    for g in range(min(NBUF, BLKS_PER_W)):
        add_block(g)
        gathers[g] = gather(g, g)
    for g in range(BLKS_PER_W):
        slot = g % NBUF
        ng = g + NBUF
        if ng < BLKS_PER_W:
            add_block(ng)
        gathers[g].wait()
        outs[g] = pltpu.async_copy(
            bufs[slot], out_hbm.at[pl.ds(base + g * BLK, BLK)], osem)
        if ng < BLKS_PER_W:
            outs[g].wait()
            gathers[ng] = gather(ng, slot)
    for g in range(max(0, BLKS_PER_W - NBUF), BLKS_PER_W):
        outs[g].wait()


@jax.jit
def _run(tables_flat, idx_t, offs):
    mesh = plsc.VectorSubcoreMesh(
        core_axis_name="c", subcore_axis_name="s", num_cores=NC,
        num_subcores=NS)
    return pl.kernel(
        _body,
        out_type=jax.ShapeDtypeStruct((TOTAL_ROWS, EMB_DIM), jnp.float32),
        mesh=mesh,
        scratch_types=[
            pltpu.VMEM((ROWS_PER_W,), jnp.int32),
            pltpu.VMEM((ROWS_PER_W,), jnp.int32),
            pltpu.VMEM((BLK, EMB_DIM), jnp.float32),
            pltpu.VMEM((BLK, EMB_DIM), jnp.float32),
            pltpu.VMEM((BLK, EMB_DIM), jnp.float32),
            pltpu.VMEM((BLK, EMB_DIM), jnp.float32),
            pltpu.VMEM((BLK, EMB_DIM), jnp.float32),
            pltpu.VMEM((BLK, EMB_DIM), jnp.float32),
            pltpu.VMEM((BLK, EMB_DIM), jnp.float32),
            pltpu.SemaphoreType.DMA,
            pltpu.SemaphoreType.DMA,
        ],
    )(tables_flat, idx_t, offs)


_OFFS = jnp.asarray(
    (np.arange(TOTAL_ROWS, dtype=np.int32) // BATCH) * VOCAB,
    dtype=jnp.int32).reshape(NW, ROWS_PER_W)


def kernel(inputs, tables):
    idx_t = inputs.T.reshape(NW, ROWS_PER_W)
    tables_flat = tables.reshape(NUM_FIELDS * VOCAB, EMB_DIM)
    out = _run(tables_flat, idx_t, _OFFS)
    return out.reshape(NUM_FIELDS, BATCH, EMB_DIM).transpose(1, 0, 2)


# speedup vs baseline: 1.0025x; 1.0025x over previous
import functools

import jax
import jax.numpy as jnp
import numpy as np
from jax import lax
from jax.experimental import pallas as pl
from jax.experimental.pallas import tpu as pltpu
from jax.experimental.pallas import tpu_sc as plsc

NUM_FIELDS = 26
VOCAB = 100000
EMB_DIM = 128
BATCH = 4096

NC, NS, LANES = 2, 16, 16
NW = NC * NS
TOTAL_ROWS = BATCH * NUM_FIELDS
ROWS_PER_W = TOTAL_ROWS // NW
BLK = 128
BLKS_PER_W = ROWS_PER_W // BLK
NBUF = 6


def _body(tab_hbm, idx_hbm, offs_hbm, out_hbm, idx_v, offs_v, rows0, rows1,
          rows2, rows3, rows4, rows5, gsem, osem):
    wid = lax.axis_index("s") * NC + lax.axis_index("c")
    base = wid * ROWS_PER_W

    c1 = pltpu.async_copy(idx_hbm.at[wid], idx_v, gsem)
    c2 = pltpu.async_copy(offs_hbm.at[wid], offs_v, gsem)
    c1.wait()
    c2.wait()

    def add_block(g):
        for c in range(BLK // LANES):
            sl = pl.ds(g * BLK + c * LANES, LANES)
            idx_v[sl] = idx_v[sl] + offs_v[sl]

    bufs = (rows0, rows1, rows2, rows3, rows4, rows5)

    def gather(g, slot):
        return pltpu.async_copy(
            tab_hbm.at[idx_v.at[pl.ds(g * BLK, BLK)]], bufs[slot], gsem)

    gathers = {}
    outs = {}
    for g in range(min(NBUF, BLKS_PER_W)):
        add_block(g)
        gathers[g] = gather(g, g)
    for g in range(BLKS_PER_W):
        slot = g % NBUF
        ng = g + NBUF
        if ng < BLKS_PER_W:
            add_block(ng)
        gathers[g].wait()
        outs[g] = pltpu.async_copy(
            bufs[slot], out_hbm.at[pl.ds(base + g * BLK, BLK)], osem)
        if ng < BLKS_PER_W:
            outs[g].wait()
            gathers[ng] = gather(ng, slot)
    for g in range(max(0, BLKS_PER_W - NBUF), BLKS_PER_W):
        outs[g].wait()


@jax.jit
def _run(tables_flat, idx_t, offs):
    mesh = plsc.VectorSubcoreMesh(
        core_axis_name="c", subcore_axis_name="s", num_cores=NC,
        num_subcores=NS)
    return pl.kernel(
        _body,
        out_type=jax.ShapeDtypeStruct((TOTAL_ROWS, EMB_DIM), jnp.float32),
        mesh=mesh,
        scratch_types=[
            pltpu.VMEM((ROWS_PER_W,), jnp.int32),
            pltpu.VMEM((ROWS_PER_W,), jnp.int32),
            pltpu.VMEM((BLK, EMB_DIM), jnp.float32),
            pltpu.VMEM((BLK, EMB_DIM), jnp.float32),
            pltpu.VMEM((BLK, EMB_DIM), jnp.float32),
            pltpu.VMEM((BLK, EMB_DIM), jnp.float32),
            pltpu.VMEM((BLK, EMB_DIM), jnp.float32),
            pltpu.VMEM((BLK, EMB_DIM), jnp.float32),
            pltpu.SemaphoreType.DMA,
            pltpu.SemaphoreType.DMA,
        ],
    )(tables_flat, idx_t, offs)


_OFFS = jnp.asarray(
    (np.arange(TOTAL_ROWS, dtype=np.int32) // BATCH) * VOCAB,
    dtype=jnp.int32).reshape(NW, ROWS_PER_W)


def kernel(inputs, tables):
    idx_t = inputs.T.reshape(NW, ROWS_PER_W)
    tables_flat = tables.reshape(NUM_FIELDS * VOCAB, EMB_DIM)
    out = _run(tables_flat, idx_t, _OFFS)
    return out.reshape(NUM_FIELDS, BATCH, EMB_DIM).transpose(1, 0, 2)
